# Initial kernel scaffold; baseline (speedup 1.0000x reference)
import jax
import jax.numpy as jnp
from jax.experimental import pallas as pl


def kernel(x, pos, edge_index, W1_0, b1_0, W2_0, b2_0, W1_1, b1_1, W2_1, b2_1, W1_2, b1_2, W2_2, b2_2, W1_3, b1_3, W2_3, b2_3, W1_4, b1_4, W2_4, b2_4):
    raise NotImplementedError("write your pallas kernel here")


# speedup vs baseline: 29.4214x; 29.4214x over previous
import functools

import jax
import jax.numpy as jnp
from jax import lax
from jax.experimental import pallas as pl
from jax.experimental.pallas import tpu as pltpu
from jax.experimental.pallas import tpu_sc as plsc

NC = 2
NS = 16
NW = NC * NS
LANES = 16
K = 128
PW = 16

_GRIDS = [(128, 96), (64, 48), (32, 24), (16, 12)]
_R_EFF = 0.025


def _align(v, m):
    return -(-v // m) * m


@functools.lru_cache(maxsize=None)
def _edge_call(n_acc, C, scale, e_pad):
    CV = C // LANES
    chunks = e_pad // (K * NW)
    rows_s = n_acc // NS
    mesh = plsc.VectorSubcoreMesh(core_axis_name="c", subcore_axis_name="s",
                                  num_cores=NC, num_subcores=NS)

    def body(src_h, dst_h, y_h, p_h, w_h, z_h, out_h,
             srcv, dstv, yr, ps, pd, msg, w0r, w1r, acc, s1, s2, s3):
        c = lax.axis_index("c")
        s = lax.axis_index("s")
        w = c * NS + s
        pltpu.sync_copy(z_h.at[pl.ds(s * rows_s, rows_s)],
                        acc.at[pl.ds(s * rows_s, rows_s)])
        pltpu.sync_copy(w_h.at[0], w0r)
        pltpu.sync_copy(w_h.at[1], w1r)
        plsc.subcore_barrier()
        w0 = [w0r[pl.ds(cv * LANES, LANES)] for cv in range(CV)]
        w1 = [w1r[pl.ds(cv * LANES, LANES)] for cv in range(CV)]

        def chunk_body(k, carry):
            off = (w * chunks + k) * K
            pltpu.sync_copy(src_h.at[pl.ds(off, K)], srcv)
            pltpu.sync_copy(dst_h.at[pl.ds(off, K)], dstv)
            cp1 = pltpu.async_copy(y_h.at[srcv], yr, s1)
            cp2 = pltpu.async_copy(p_h.at[srcv], ps, s2)
            cp3 = pltpu.async_copy(p_h.at[dstv], pd, s3)
            cp1.wait()
            cp2.wait()
            cp3.wait()

            def edge_body(e, carry2):
                eav = jnp.clip((pd[e, :] - ps[e, :]) * scale + 0.5, 0.0, 1.0)
                a0 = jnp.full((LANES,), eav[0], jnp.float32)
                a1 = jnp.full((LANES,), eav[1], jnp.float32)
                for cv in range(CV):
                    yv = yr[e, pl.ds(cv * LANES, LANES)]
                    m = jnp.maximum(yv + a0 * w0[cv] + a1 * w1[cv], 0.0)
                    msg[e, pl.ds(cv * LANES, LANES)] = m
                return carry2

            lax.fori_loop(0, K, edge_body, 0)
            pltpu.sync_copy(msg, acc.at[dstv], add=True)
            return carry

        lax.fori_loop(0, chunks, chunk_body, 0)
        plsc.subcore_barrier()
        pltpu.sync_copy(acc.at[pl.ds(s * rows_s, rows_s)],
                        out_h.at[c, pl.ds(s * rows_s, rows_s)])

    return pl.kernel(
        body,
        out_type=jax.ShapeDtypeStruct((NC, n_acc, C), jnp.float32),
        mesh=mesh,
        compiler_params=pltpu.CompilerParams(use_tc_tiling_on_sc=False),
        scratch_types=[
            pltpu.VMEM((K,), jnp.int32),
            pltpu.VMEM((K,), jnp.int32),
            pltpu.VMEM((K, C), jnp.float32),
            pltpu.VMEM((K, PW), jnp.float32),
            pltpu.VMEM((K, PW), jnp.float32),
            pltpu.VMEM((K, C), jnp.float32),
            pltpu.VMEM((C,), jnp.float32),
            pltpu.VMEM((C,), jnp.float32),
            pltpu.VMEM_SHARED((n_acc, C), jnp.float32),
            pltpu.SemaphoreType.DMA,
            pltpu.SemaphoreType.DMA,
            pltpu.SemaphoreType.DMA,
        ],
    )


@functools.lru_cache(maxsize=None)
def _remap_call(n_clp, e_pad):
    chunks = e_pad // (K * NW)
    mesh = plsc.VectorSubcoreMesh(core_axis_name="c", subcore_axis_name="s",
                                  num_cores=NC, num_subcores=NS)

    def body(cl_h, a_h, b_h, oa_h, ob_h, clv, av, ov):
        c = lax.axis_index("c")
        s = lax.axis_index("s")
        w = c * NS + s
        pltpu.sync_copy(cl_h, clv)
        iota = lax.iota(jnp.int32, LANES)

        def chunk_body(k, carry):
            off = (w * chunks + k) * K
            for in_h, out_h in ((a_h, oa_h), (b_h, ob_h)):
                pltpu.sync_copy(in_h.at[pl.ds(off, K)], av)

                def group_body(g, carry2):
                    vals = av[pl.ds(g * LANES, LANES)]
                    ov[pl.ds(g * LANES, LANES)] = plsc.load_gather(clv, [vals])
                    return carry2

                lax.fori_loop(0, K // LANES, group_body, 0)
                pltpu.sync_copy(ov, out_h.at[pl.ds(off, K)])
            return carry

        lax.fori_loop(0, chunks, chunk_body, 0)

    return pl.kernel(
        body,
        out_type=(jax.ShapeDtypeStruct((e_pad,), jnp.int32),
                  jax.ShapeDtypeStruct((e_pad,), jnp.int32)),
        mesh=mesh,
        compiler_params=pltpu.CompilerParams(use_tc_tiling_on_sc=False,
                                             needs_layout_passes=False),
        scratch_types=[
            pltpu.VMEM((n_clp,), jnp.int32),
            pltpu.VMEM((K,), jnp.int32),
            pltpu.VMEM((K,), jnp.int32),
        ],
    )


def _sc_layer(feats, p2, srcp, dstp, n, scale, W1, b1, W2, b2):
    cin = feats.shape[1]
    C = W1.shape[1]
    n_acc = _align(n + 1, NS * 8)
    Y = feats @ W1[:cin] + b1
    Yp = jnp.concatenate([Y, jnp.zeros((1, C), jnp.float32)], axis=0)
    P = jnp.pad(p2, ((0, 1), (0, PW - 2)))
    w01 = W1[cin:cin + 2]
    zeros = jnp.zeros((n_acc, C), jnp.float32)
    out = _edge_call(n_acc, C, float(scale), srcp.shape[0])(
        srcp, dstp, Yp, P, w01, zeros)
    agg = (out[0] + out[1])[:n]
    return jax.nn.relu(agg @ W2 + b2)


def _voxel_pool(h, pos, gx, gy, aggr):
    n = gx * gy
    ix = jnp.clip((pos[:, 0] * gx).astype(jnp.int32), 0, gx - 1)
    iy = jnp.clip((pos[:, 1] * gy).astype(jnp.int32), 0, gy - 1)
    cl = (iy * gx + ix).astype(jnp.int32)
    cnt = jax.ops.segment_sum(jnp.ones((pos.shape[0],), jnp.float32), cl,
                              num_segments=n)
    if aggr == 'max':
        xf = jax.ops.segment_max(h, cl, num_segments=n)
        xf = jnp.where(cnt[:, None] > 0, xf, 0.0)
    else:
        xf = jax.ops.segment_sum(h, cl, num_segments=n) \
            / jnp.maximum(cnt, 1.0)[:, None]
    pn = jax.ops.segment_sum(pos, cl, num_segments=n) \
        / jnp.maximum(cnt, 1.0)[:, None]
    return xf, pn, cl


def kernel(x, pos, edge_index,
           W1_0, b1_0, W2_0, b2_0,
           W1_1, b1_1, W2_1, b2_1,
           W1_2, b1_2, W2_2, b2_2,
           W1_3, b1_3, W2_3, b2_3,
           W1_4, b1_4, W2_4, b2_4):
    n = x.shape[0]
    E = edge_index.shape[1]
    e_pad = _align(E, K * NW)
    pad = e_pad - E
    srcp = jnp.concatenate(
        [edge_index[0], jnp.zeros((pad,), jnp.int32)])
    dstp = jnp.concatenate(
        [edge_index[1], jnp.full((pad,), n, jnp.int32)])

    feats = jnp.concatenate([x, pos[:, :2]], axis=1)
    h = _sc_layer(feats, pos[:, :2], srcp, dstp, n,
                  1.0 / (2.0 * _R_EFF), W1_0, b1_0, W2_0, b2_0)

    maxvs = [2.0 * _R_EFF, 2.0 / 48.0, 2.0 / 24.0, 2.0 / 12.0]
    Ws = [(W1_1, b1_1, W2_1, b2_1), (W1_2, b1_2, W2_2, b2_2),
          (W1_3, b1_3, W2_3, b2_3), (W1_4, b1_4, W2_4, b2_4)]
    aggrs = ['max', 'max', 'max', 'mean']
    pos_cur = pos
    for i in range(4):
        gx, gy = _GRIDS[i]
        n_next = gx * gy
        xf, pn, cl = _voxel_pool(h, pos_cur, gx, gy, aggrs[i])
        n_clp = _align(n + 1, 8)
        clp = jnp.concatenate(
            [cl, jnp.full((n_clp - n,), n_next, jnp.int32)])
        srcp, dstp = _remap_call(n_clp, e_pad)(clp, srcp, dstp)
        feats = jnp.concatenate([xf, pn[:, :2]], axis=1)
        W1, b1, W2, b2 = Ws[i]
        h = _sc_layer(feats, pn[:, :2], srcp, dstp, n_next,
                      1.0 / (2.0 * maxvs[i]), W1, b1, W2, b2)
        pos_cur = pn
        n = n_next
    return h
